# Initial kernel scaffold; baseline (speedup 1.0000x reference)
import jax
import jax.numpy as jnp
from jax.experimental import pallas as pl


def kernel(word_ids, topic_ids, emb_word, emb_topics, W1, b1, W2, b2):
    raise NotImplementedError("write your pallas kernel here")


# speedup vs baseline: 12.4140x; 12.4140x over previous
import functools

import jax
import jax.numpy as jnp
from jax import lax
from jax.experimental import pallas as pl
from jax.experimental.pallas import tpu as pltpu
from jax.experimental.pallas import tpu_sc as plsc

B, L = 4096, 200
EMBED, TOPICS = 64, 64
D = EMBED + TOPICS
HIDDEN, NUM_CLASSES = 256, 10

NC, NS = 2, 16
NW = NC * NS
B_PER_W = B // NW

CHUNKS = ((0, 128), (128, 72))


def _pooled_sc_kernel(word_ids_hbm, topic_ids_hbm, emb_word_hbm,
                      emb_topics_hbm, out_hbm,
                      idxw, idxt, wrows, trows, outv, sem):
    wid = lax.axis_index("s") * NC + lax.axis_index("c")
    base = wid * B_PER_W

    def row_body(b, carry):
        row = base + b
        pltpu.sync_copy(word_ids_hbm.at[pl.ds(row, 1)], idxw)
        pltpu.sync_copy(topic_ids_hbm.at[pl.ds(row, 1)], idxt)
        copies = []
        for off, n in CHUNKS:
            copies.append(pltpu.async_copy(
                emb_word_hbm.at[idxw.at[0, pl.ds(off, n)]],
                wrows.at[pl.ds(off, n)], sem))
            copies.append(pltpu.async_copy(
                emb_topics_hbm.at[idxt.at[0, pl.ds(off, n)]],
                trows.at[pl.ds(off, n)], sem))
        for c in copies:
            c.wait()

        def red(l, acc):
            new = []
            for j in range(4):
                new.append(acc[j] + wrows[l, pl.ds(16 * j, 16)])
            for j in range(4):
                new.append(acc[4 + j] + trows[l, pl.ds(16 * j, 16)])
            return tuple(new)

        zeros = tuple(jnp.zeros((16,), jnp.float32) for _ in range(8))
        acc = lax.fori_loop(0, L, red, zeros, unroll=2)
        scale = jnp.float32(1.0 / L)
        for j in range(4):
            outv[b, pl.ds(16 * j, 16)] = acc[j] * scale
        for j in range(4):
            outv[b, pl.ds(EMBED + 16 * j, 16)] = acc[4 + j] * scale
        return carry

    lax.fori_loop(0, B_PER_W, row_body, 0)
    pltpu.sync_copy(outv, out_hbm.at[pl.ds(base, B_PER_W)])


def _pooled(word_ids, topic_ids, emb_word, emb_topics):
    mesh = plsc.VectorSubcoreMesh(core_axis_name="c", subcore_axis_name="s",
                                  num_cores=NC, num_subcores=NS)
    f = pl.kernel(
        _pooled_sc_kernel,
        out_type=jax.ShapeDtypeStruct((B, D), jnp.float32),
        mesh=mesh,
        scratch_types=[
            pltpu.VMEM((1, L), jnp.int32),
            pltpu.VMEM((1, L), jnp.int32),
            pltpu.VMEM((L, EMBED), jnp.float32),
            pltpu.VMEM((L, TOPICS), jnp.float32),
            pltpu.VMEM((B_PER_W, D), jnp.float32),
            pltpu.SemaphoreType.DMA,
        ],
        compiler_params=pltpu.CompilerParams(use_tc_tiling_on_sc=False),
    )
    return f(word_ids, topic_ids, emb_word, emb_topics)


def _mlp_kernel(x_ref, w1_ref, b1_ref, w2_ref, b2_ref, o_ref):
    h = jnp.dot(x_ref[...], w1_ref[...], preferred_element_type=jnp.float32)
    h = jnp.maximum(h + b1_ref[...], 0.0)
    o = jnp.dot(h, w2_ref[...], preferred_element_type=jnp.float32)
    o_ref[...] = o + b2_ref[...]


def _mlp(pooled, W1, b1, W2, b2):
    W2p = jnp.zeros((HIDDEN, 128), jnp.float32).at[:, :NUM_CLASSES].set(W2)
    b2p = jnp.zeros((1, 128), jnp.float32).at[0, :NUM_CLASSES].set(b2)
    out = pl.pallas_call(
        _mlp_kernel,
        out_shape=jax.ShapeDtypeStruct((B, 128), jnp.float32),
    )(pooled, W1, b1.reshape(1, HIDDEN), W2p, b2p)
    return out[:, :NUM_CLASSES]


@jax.jit
def kernel(word_ids, topic_ids, emb_word, emb_topics, W1, b1, W2, b2):
    word_ids = word_ids.astype(jnp.int32)
    topic_ids = topic_ids.astype(jnp.int32)
    pooled = _pooled(word_ids, topic_ids, emb_word, emb_topics)
    return _mlp(pooled, W1, b1, W2, b2)


# speedup vs baseline: 18.7839x; 1.5131x over previous
import functools

import jax
import jax.numpy as jnp
from jax import lax
from jax.experimental import pallas as pl
from jax.experimental.pallas import tpu as pltpu
from jax.experimental.pallas import tpu_sc as plsc

B, L = 4096, 200
EMBED, TOPICS = 64, 64
D = EMBED + TOPICS
HIDDEN, NUM_CLASSES = 256, 10

NC, NS = 2, 16
NW = NC * NS
B_PER_W = B // NW
IB = 16
NBLK = B_PER_W // IB

CHUNKS = ((0, 128), (128, 72))


def _pooled_sc_kernel(word_ids_hbm, topic_ids_hbm, emb_word_hbm,
                      emb_topics_hbm, out_hbm,
                      idxw, idxt, wrA, wrB, trA, trB, outv, semA, semB):
    wid = lax.axis_index("s") * NC + lax.axis_index("c")
    base = wid * B_PER_W
    wr = (wrA, wrB)
    tr = (trA, trB)
    sems = (semA, semB)

    def issue(r, slot):
        copies = []
        for off, n in CHUNKS:
            copies.append(pltpu.async_copy(
                emb_word_hbm.at[idxw.at[r, pl.ds(off, n)]],
                wr[slot].at[pl.ds(off, n)], sems[slot]))
            copies.append(pltpu.async_copy(
                emb_topics_hbm.at[idxt.at[r, pl.ds(off, n)]],
                tr[slot].at[pl.ds(off, n)], sems[slot]))
        return copies

    def reduce_row(slot, r_out):
        def red(l, acc):
            new = []
            for j in range(4):
                new.append(acc[j] + wr[slot][l, pl.ds(16 * j, 16)])
            for j in range(4):
                new.append(acc[4 + j] + tr[slot][l, pl.ds(16 * j, 16)])
            return tuple(new)

        zeros = tuple(jnp.zeros((16,), jnp.float32) for _ in range(8))
        acc = lax.fori_loop(0, L, red, zeros, unroll=2)
        scale = jnp.float32(1.0 / L)
        for j in range(4):
            outv[r_out, pl.ds(16 * j, 16)] = acc[j] * scale
        for j in range(4):
            outv[r_out, pl.ds(EMBED + 16 * j, 16)] = acc[4 + j] * scale

    def block_body(k, carry):
        row0 = base + k * IB
        pltpu.sync_copy(word_ids_hbm.at[pl.ds(row0, IB)], idxw)
        pltpu.sync_copy(topic_ids_hbm.at[pl.ds(row0, IB)], idxt)
        pending = issue(0, 0)
        for r in range(IB):
            cur = r % 2
            for c in pending:
                c.wait()
            if r + 1 < IB:
                pending = issue(r + 1, 1 - cur)
            reduce_row(cur, r)
        pltpu.sync_copy(outv, out_hbm.at[pl.ds(row0, IB)])
        return carry

    lax.fori_loop(0, NBLK, block_body, 0)


def _pooled(word_ids, topic_ids, emb_word, emb_topics):
    mesh = plsc.VectorSubcoreMesh(core_axis_name="c", subcore_axis_name="s",
                                  num_cores=NC, num_subcores=NS)
    f = pl.kernel(
        _pooled_sc_kernel,
        out_type=jax.ShapeDtypeStruct((B, D), jnp.float32),
        mesh=mesh,
        scratch_types=[
            pltpu.VMEM((IB, L), jnp.int32),
            pltpu.VMEM((IB, L), jnp.int32),
            pltpu.VMEM((L, EMBED), jnp.float32),
            pltpu.VMEM((L, EMBED), jnp.float32),
            pltpu.VMEM((L, TOPICS), jnp.float32),
            pltpu.VMEM((L, TOPICS), jnp.float32),
            pltpu.VMEM((IB, D), jnp.float32),
            pltpu.SemaphoreType.DMA,
            pltpu.SemaphoreType.DMA,
        ],
        compiler_params=pltpu.CompilerParams(use_tc_tiling_on_sc=False),
    )
    return f(word_ids, topic_ids, emb_word, emb_topics)


def _mlp_kernel(x_ref, w1_ref, b1_ref, w2_ref, b2_ref, o_ref):
    h = jnp.dot(x_ref[...], w1_ref[...], preferred_element_type=jnp.float32)
    h = jnp.maximum(h + b1_ref[...], 0.0)
    o = jnp.dot(h, w2_ref[...], preferred_element_type=jnp.float32)
    o_ref[...] = o + b2_ref[...]


def _mlp(pooled, W1, b1, W2, b2):
    W2p = jnp.zeros((HIDDEN, 128), jnp.float32).at[:, :NUM_CLASSES].set(W2)
    b2p = jnp.zeros((1, 128), jnp.float32).at[0, :NUM_CLASSES].set(b2)
    out = pl.pallas_call(
        _mlp_kernel,
        out_shape=jax.ShapeDtypeStruct((B, 128), jnp.float32),
    )(pooled, W1, b1.reshape(1, HIDDEN), W2p, b2p)
    return out[:, :NUM_CLASSES]


@jax.jit
def kernel(word_ids, topic_ids, emb_word, emb_topics, W1, b1, W2, b2):
    word_ids = word_ids.astype(jnp.int32)
    topic_ids = topic_ids.astype(jnp.int32)
    pooled = _pooled(word_ids, topic_ids, emb_word, emb_topics)
    return _mlp(pooled, W1, b1, W2, b2)
